# Initial kernel scaffold; baseline (speedup 1.0000x reference)
import jax
import jax.numpy as jnp
from jax.experimental import pallas as pl


def kernel(x, edge_index, batch, params):
    raise NotImplementedError("write your pallas kernel here")


# speedup vs baseline: 3.2407x; 3.2407x over previous
import functools

import jax
import jax.numpy as jnp
from jax import lax
from jax.experimental import pallas as pl
from jax.experimental.pallas import tpu as pltpu
from jax.experimental.pallas import tpu_sc as plsc

_NC = 2
_NS = 16
_BE = 80
_BLK = 1000
_G = 64


def _sc_segment_sum(h2, src, dst, zz, split_edges):
    W = h2.shape[1]
    NP = zz.shape[0]
    N = h2.shape[0] if split_edges else h2.shape[0] // 2
    E = src.shape[0]
    ept = E // (_NS * (2 if split_edges else 1))
    nchunk = ept // _BE
    rpt = NP // _NS

    mesh = plsc.VectorSubcoreMesh(
        core_axis_name="c", subcore_axis_name="s", num_cores=_NC, num_subcores=_NS
    )

    @functools.partial(
        pl.kernel,
        out_type=jax.ShapeDtypeStruct((2 * NP, W), jnp.float32),
        mesh=mesh,
        scratch_types=[
            pltpu.VMEM_SHARED((NP, W), jnp.float32),
            pltpu.VMEM((_BE,), jnp.int32),
            pltpu.VMEM((_BE,), jnp.int32),
            pltpu.VMEM((_BE, W), jnp.float32),
            pltpu.SemaphoreType.DMA,
        ],
    )
    def k(h_hbm, src_hbm, dst_hbm, zz_hbm, out_hbm, accum, src_v, dst_v, rows_v, sem):
        c = lax.axis_index("c")
        s = lax.axis_index("s")
        pltpu.sync_copy(zz_hbm.at[pl.ds(s * rpt, rpt)], accum.at[pl.ds(s * rpt, rpt)])
        plsc.subcore_barrier()

        ooff = c * NP
        if split_edges:
            ebase = c * (E // 2) + s * ept
        else:
            ebase = s * ept
            roff = c * N

        def chunk(i, carry):
            off = ebase + i * _BE
            pltpu.sync_copy(src_hbm.at[pl.ds(off, _BE)], src_v)
            pltpu.sync_copy(dst_hbm.at[pl.ds(off, _BE)], dst_v)
            if not split_edges:
                for kk in range(_BE // 16):
                    sl = pl.ds(kk * 16, 16)
                    src_v[sl] = src_v[sl] + roff
            pltpu.async_copy(h_hbm.at[src_v], rows_v, sem).wait()
            pltpu.sync_copy(rows_v, accum.at[dst_v], add=True)
            return carry

        lax.fori_loop(0, nchunk, chunk, 0)
        plsc.subcore_barrier()
        pltpu.sync_copy(
            accum.at[pl.ds(s * rpt, rpt)], out_hbm.at[pl.ds(ooff + s * rpt, rpt)]
        )

    return k(h2, src, dst, zz)


def _dot(a, b, dims):
    return lax.dot_general(
        a,
        b,
        dims,
        preferred_element_type=jnp.float32,
        precision=lax.Precision.HIGHEST,
    )


def _tc_mlp_stats(h_cols, agg_cols, w1, b1, w2, b2, sum_agg):
    hc, N, W = h_cols.shape
    H = w2.shape[1]
    nblk = N // _BLK

    def body(h_ref, a_ref, w1_ref, b1_ref, w2_ref, b2_ref, z_ref, st_ref):
        i = pl.program_id(0)
        if hc == 1:
            h = h_ref[0]
        else:
            h = jnp.concatenate([h_ref[0], h_ref[1]], axis=1)
        if sum_agg:
            a = a_ref[0] + a_ref[1]
        else:
            a = jnp.concatenate([a_ref[0], a_ref[1]], axis=1)
        z = h + a
        z1 = jnp.maximum(
            _dot(z, w1_ref[...], (((1,), (0,)), ((), ()))) + b1_ref[...], 0.0
        )
        z2 = _dot(z1, w2_ref[...], (((1,), (0,)), ((), ()))) + b2_ref[...]
        z_ref[...] = z2
        s1 = jnp.sum(z2, axis=0, keepdims=True)
        s2 = jnp.sum(z2 * z2, axis=0, keepdims=True)
        contrib = jnp.concatenate(
            [s1, s2, jnp.zeros((6, H), jnp.float32)], axis=0
        )

        @pl.when(i == 0)
        def _():
            st_ref[...] = contrib

        @pl.when(i > 0)
        def _():
            st_ref[...] = st_ref[...] + contrib

    return pl.pallas_call(
        body,
        grid=(nblk,),
        in_specs=[
            pl.BlockSpec((hc, _BLK, W), lambda i: (0, i, 0)),
            pl.BlockSpec((2, _BLK, W), lambda i: (0, i, 0)),
            pl.BlockSpec((w1.shape[0], H), lambda i: (0, 0)),
            pl.BlockSpec((1, H), lambda i: (0, 0)),
            pl.BlockSpec((H, H), lambda i: (0, 0)),
            pl.BlockSpec((1, H), lambda i: (0, 0)),
        ],
        out_specs=[
            pl.BlockSpec((_BLK, H), lambda i: (i, 0)),
            pl.BlockSpec((8, H), lambda i: (0, 0)),
        ],
        out_shape=[
            jax.ShapeDtypeStruct((N, H), jnp.float32),
            jax.ShapeDtypeStruct((8, H), jnp.float32),
        ],
    )(h_cols, agg_cols, w1, b1, w2, b2)


def _tc_norm_pool(z2, stats, gamma, beta, batch_f):
    N, H = z2.shape
    Wh = H // 2
    nblk = N // _BLK
    inv_n = 1.0 / N

    def body(z_ref, st_ref, g_ref, b_ref, bt_ref, h_ref, p_ref):
        i = pl.program_id(0)
        mu = st_ref[0:1, :] * inv_n
        ex2 = st_ref[1:2, :] * inv_n
        var = ex2 - mu * mu
        inv = lax.rsqrt(var + 1e-5)
        hn = jnp.maximum((z_ref[...] - mu) * inv * g_ref[...] + b_ref[...], 0.0)
        h_ref[0] = hn[:, :Wh]
        h_ref[1] = hn[:, Wh:]
        gids = lax.broadcasted_iota(jnp.int32, (1, _G), 1).astype(jnp.float32)
        onehot = jnp.where(bt_ref[...] == gids, 1.0, 0.0)
        contrib = _dot(onehot, hn, (((0,), (0,)), ((), ())))

        @pl.when(i == 0)
        def _():
            p_ref[...] = contrib

        @pl.when(i > 0)
        def _():
            p_ref[...] = p_ref[...] + contrib

    return pl.pallas_call(
        body,
        grid=(nblk,),
        in_specs=[
            pl.BlockSpec((_BLK, H), lambda i: (i, 0)),
            pl.BlockSpec((8, H), lambda i: (0, 0)),
            pl.BlockSpec((1, H), lambda i: (0, 0)),
            pl.BlockSpec((1, H), lambda i: (0, 0)),
            pl.BlockSpec((_BLK, 1), lambda i: (i, 0)),
        ],
        out_specs=[
            pl.BlockSpec((2, _BLK, Wh), lambda i: (0, i, 0)),
            pl.BlockSpec((_G, H), lambda i: (0, 0)),
        ],
        out_shape=[
            jax.ShapeDtypeStruct((2, N, Wh), jnp.float32),
            jax.ShapeDtypeStruct((_G, H), jnp.float32),
        ],
    )(z2, stats, gamma, beta, batch_f)


def _tc_pool_x(x, batch_f):
    N, D = x.shape
    nblk = N // _BLK

    def body(x_ref, bt_ref, p_ref, c_ref):
        i = pl.program_id(0)
        gids = lax.broadcasted_iota(jnp.int32, (1, _G), 1).astype(jnp.float32)
        onehot = jnp.where(bt_ref[...] == gids, 1.0, 0.0)
        contrib = _dot(onehot, x_ref[...], (((0,), (0,)), ((), ())))
        ones = jnp.ones((_BLK, 8), jnp.float32)
        cnt = _dot(onehot, ones, (((0,), (0,)), ((), ())))

        @pl.when(i == 0)
        def _():
            p_ref[...] = contrib
            c_ref[...] = cnt

        @pl.when(i > 0)
        def _():
            p_ref[...] = p_ref[...] + contrib
            c_ref[...] = c_ref[...] + cnt

    return pl.pallas_call(
        body,
        grid=(nblk,),
        in_specs=[
            pl.BlockSpec((_BLK, D), lambda i: (i, 0)),
            pl.BlockSpec((_BLK, 1), lambda i: (i, 0)),
        ],
        out_specs=[
            pl.BlockSpec((_G, D), lambda i: (0, 0)),
            pl.BlockSpec((_G, 8), lambda i: (0, 0)),
        ],
        out_shape=[
            jax.ShapeDtypeStruct((_G, D), jnp.float32),
            jax.ShapeDtypeStruct((_G, 8), jnp.float32),
        ],
    )(x, batch_f)


def _tc_head(pooled_list, counts, pred_ws, pred_bs, wf, bf, wc, bc):
    G = pooled_list[0].shape[0]
    H = wf.shape[0]
    C = wc.shape[1]
    n_pred = len(pooled_list)

    def body(*refs):
        pooled_refs = refs[:n_pred]
        c_ref = refs[n_pred]
        w_refs = refs[n_pred + 1 : 2 * n_pred + 1]
        b_refs = refs[2 * n_pred + 1 : 3 * n_pred + 1]
        wf_ref, bf_ref, wc_ref, bc_ref, out_ref = refs[3 * n_pred + 1 :]
        cnt = jnp.maximum(c_ref[:, 0:1], 1.0)
        score = jnp.zeros((G, H), jnp.float32)
        for j in range(n_pred):
            pooled = pooled_refs[j][...] / cnt
            score = score + _dot(pooled, w_refs[j][...], (((1,), (0,)), ((), ())))
            score = score + b_refs[j][...]
        f = jnp.maximum(_dot(score, wf_ref[...], (((1,), (0,)), ((), ()))) + bf_ref[...], 0.0)
        out_ref[...] = _dot(f, wc_ref[...], (((1,), (0,)), ((), ()))) + bc_ref[...]

    args = list(pooled_list) + [counts] + list(pred_ws) + list(pred_bs) + [wf, bf, wc, bc]
    return pl.pallas_call(
        body,
        out_shape=jax.ShapeDtypeStruct((G, C), jnp.float32),
    )(*args)


def kernel(x, edge_index, batch, params):
    x = x.astype(jnp.float32)
    src = edge_index[0]
    dst = edge_index[1]
    N, D = x.shape
    L = len(params["convs"])
    batch_f = batch.astype(jnp.float32)[:, None]

    pooled = [None] * (L + 1)
    pooled[0], counts = _tc_pool_x(x, batch_f)

    NP = ((N + _NS * 8 - 1) // (_NS * 8)) * (_NS * 8)
    h_cols = x[None]

    for i in range(L):
        c = params["convs"][i]
        bn = params["bns"][i]
        split_edges = h_cols.shape[0] == 1
        W = h_cols.shape[2]
        zz = jnp.zeros((NP, W), jnp.float32)
        h2 = h_cols[0] if split_edges else h_cols.reshape(2 * N, W)
        agg2 = _sc_segment_sum(h2, src, dst, zz, split_edges)
        z2, stats = _tc_mlp_stats(
            h_cols,
            agg2.reshape(2, NP, W),
            c["W1"],
            c["b1"][None, :],
            c["W2"],
            c["b2"][None, :],
            sum_agg=split_edges,
        )
        h_cols, pooled[i + 1] = _tc_norm_pool(
            z2, stats, bn["gamma"][None, :], bn["beta"][None, :], batch_f
        )

    pred_ws = [p["W"] for p in params["preds"]]
    pred_bs = [p["b"][None, :] for p in params["preds"]]
    return _tc_head(
        pooled,
        counts,
        pred_ws,
        pred_bs,
        params["Wf"],
        params["bf"][None, :],
        params["Wc"],
        params["bc"][None, :],
    )
